# Initial kernel scaffold; baseline (speedup 1.0000x reference)
import jax
import jax.numpy as jnp
from jax.experimental import pallas as pl


def kernel(x, W_router, W_gate, W_up, W_down):
    raise NotImplementedError("write your pallas kernel here")


# speedup vs baseline: 1.3649x; 1.3649x over previous
import functools

import jax
import jax.numpy as jnp
from jax.experimental import pallas as pl
from jax.experimental.pallas import tpu as pltpu

DIM = 768
NUM_EXPERTS = 64
HIDDEN = 2 * DIM


def _router_kernel(x_ref, wr_ref, idx_ref, wts_ref, act_ref):
    x = x_ref[...]
    wr = wr_ref[...]
    logits = jax.lax.dot_general(
        x, wr, (((1,), (1,)), ((), ())),
        preferred_element_type=jnp.float32)
    m = jnp.max(logits, axis=1, keepdims=True)
    p = jnp.exp(logits - m)
    probs = p / jnp.sum(p, axis=1, keepdims=True)

    w1 = jnp.max(probs, axis=1)
    i1 = jnp.argmax(probs, axis=1).astype(jnp.int32)
    eids = jax.lax.broadcasted_iota(jnp.int32, probs.shape, 1)
    masked = jnp.where(eids == i1[:, None], -jnp.inf, probs)
    w2 = jnp.max(masked, axis=1)
    i2 = jnp.argmax(masked, axis=1).astype(jnp.int32)
    tot = w1 + w2
    idx_ref[0, :] = i1
    idx_ref[1, :] = i2
    wts_ref[0, :] = w1 / tot
    wts_ref[1, :] = w2 / tot

    hit = (i1[:, None] == eids) | (i2[:, None] == eids)
    act_ref[...] = jnp.any(hit, axis=0).astype(jnp.int32)[None, :]


def _moe_kernel(eid_ref, na_ref, x_ref, idx_ref, wts_ref,
                wg_ref, wu_ref, wd_ref, out_ref):
    i = pl.program_id(0)
    na = na_ref[0]

    @pl.when(i < na)
    def _():
        e = eid_ref[i]
        x = x_ref[...]
        gate_h = jax.lax.dot_general(
            x, wg_ref[0], (((1,), (1,)), ((), ())),
            preferred_element_type=jnp.float32)
        up_h = jax.lax.dot_general(
            x, wu_ref[0], (((1,), (1,)), ((), ())),
            preferred_element_type=jnp.float32)
        h = (gate_h * jax.nn.sigmoid(gate_h)) * up_h
        o = jax.lax.dot_general(
            h, wd_ref[0], (((1,), (1,)), ((), ())),
            preferred_element_type=jnp.float32)
        g2 = jnp.where(idx_ref[...] == e, wts_ref[...], 0.0)
        gate = g2[0, :] + g2[1, :]
        contrib = o * gate[:, None]

        @pl.when(i == 0)
        def _():
            out_ref[...] = contrib

        @pl.when(i > 0)
        def _():
            out_ref[...] += contrib


@jax.jit
def kernel(x, W_router, W_gate, W_up, W_down):
    orig_shape = x.shape
    x2 = x.reshape(-1, DIM)
    n = x2.shape[0]

    idx, wts, act = pl.pallas_call(
        _router_kernel,
        out_shape=(
            jax.ShapeDtypeStruct((2, n), jnp.int32),
            jax.ShapeDtypeStruct((2, n), jnp.float32),
            jax.ShapeDtypeStruct((1, NUM_EXPERTS), jnp.int32),
        ),
    )(x2, W_router)

    active = act[0] > 0
    order = jnp.argsort(jnp.logical_not(active), stable=True).astype(jnp.int32)
    na = jnp.sum(active.astype(jnp.int32))
    steps = jnp.arange(NUM_EXPERTS, dtype=jnp.int32)
    eid = order[jnp.minimum(steps, na - 1)]

    grid_spec = pltpu.PrefetchScalarGridSpec(
        num_scalar_prefetch=2,
        grid=(NUM_EXPERTS,),
        in_specs=[
            pl.BlockSpec((n, DIM), lambda i, eid, na: (0, 0)),
            pl.BlockSpec((2, n), lambda i, eid, na: (0, 0)),
            pl.BlockSpec((2, n), lambda i, eid, na: (0, 0)),
            pl.BlockSpec((1, HIDDEN, DIM), lambda i, eid, na: (eid[i], 0, 0)),
            pl.BlockSpec((1, HIDDEN, DIM), lambda i, eid, na: (eid[i], 0, 0)),
            pl.BlockSpec((1, DIM, HIDDEN), lambda i, eid, na: (eid[i], 0, 0)),
        ],
        out_specs=pl.BlockSpec((n, DIM), lambda i, eid, na: (0, 0)),
    )
    out = pl.pallas_call(
        _moe_kernel,
        grid_spec=grid_spec,
        out_shape=jax.ShapeDtypeStruct((n, DIM), jnp.float32),
        compiler_params=pltpu.CompilerParams(
            dimension_semantics=("arbitrary",),
        ),
    )(eid, na.reshape(1), x2, idx, wts, W_gate, W_up, W_down)

    return out.reshape(orig_shape)


# speedup vs baseline: 1.3992x; 1.0251x over previous
import functools

import jax
import jax.numpy as jnp
from jax.experimental import pallas as pl
from jax.experimental.pallas import tpu as pltpu

DIM = 768
NUM_EXPERTS = 64
HIDDEN = 2 * DIM


def _router_kernel(x_ref, wr_ref, idx_ref, wts_ref, eid_ref, na_ref):
    x = x_ref[...]
    wr = wr_ref[...]
    logits = jax.lax.dot_general(
        x, wr, (((1,), (1,)), ((), ())),
        preferred_element_type=jnp.float32)
    m = jnp.max(logits, axis=1, keepdims=True)
    p = jnp.exp(logits - m)
    probs = p / jnp.sum(p, axis=1, keepdims=True)

    w1 = jnp.max(probs, axis=1)
    i1 = jnp.argmax(probs, axis=1).astype(jnp.int32)
    eids = jax.lax.broadcasted_iota(jnp.int32, probs.shape, 1)
    masked = jnp.where(eids == i1[:, None], -jnp.inf, probs)
    w2 = jnp.max(masked, axis=1)
    i2 = jnp.argmax(masked, axis=1).astype(jnp.int32)
    tot = w1 + w2
    idx_ref[0, :] = i1
    idx_ref[1, :] = i2
    wts_ref[0, :] = w1 / tot
    wts_ref[1, :] = w2 / tot

    hit = (i1[:, None] == eids) | (i2[:, None] == eids)
    active = jnp.any(hit, axis=0)
    E = NUM_EXPERTS
    rows = jax.lax.broadcasted_iota(jnp.int32, (E, E), 0)
    cols = jax.lax.broadcasted_iota(jnp.int32, (E, E), 1)
    tri = (rows <= cols).astype(jnp.float32)
    csum = jnp.dot(active.astype(jnp.float32)[None, :], tri,
                   preferred_element_type=jnp.float32)
    ci = csum.astype(jnp.int32)
    na = jnp.sum(active.astype(jnp.int32))
    tgt = jnp.minimum(rows + 1, na)
    sel = (ci == tgt) & active[None, :]
    eid_ref[...] = jnp.argmax(sel.astype(jnp.float32), axis=1
                              ).astype(jnp.int32)[None, :]
    na_ref[...] = na.reshape(1, 1)


def _moe_kernel(eid_ref, na_ref, x_ref, idx_ref, wts_ref,
                wg_ref, wu_ref, wd_ref, out_ref):
    i = pl.program_id(0)
    na = na_ref[0, 0]

    @pl.when(i < na)
    def _():
        e = eid_ref[0, i]
        x = x_ref[...]
        gate_h = jax.lax.dot_general(
            x, wg_ref[0], (((1,), (1,)), ((), ())),
            preferred_element_type=jnp.float32)
        up_h = jax.lax.dot_general(
            x, wu_ref[0], (((1,), (1,)), ((), ())),
            preferred_element_type=jnp.float32)
        h = (gate_h * jax.nn.sigmoid(gate_h)) * up_h
        o = jax.lax.dot_general(
            h, wd_ref[0], (((1,), (1,)), ((), ())),
            preferred_element_type=jnp.float32)
        g2 = jnp.where(idx_ref[...] == e, wts_ref[...], 0.0)
        gate = g2[0, :] + g2[1, :]
        contrib = o * gate[:, None]

        @pl.when(i == 0)
        def _():
            out_ref[...] = contrib

        @pl.when(i > 0)
        def _():
            out_ref[...] += contrib


@jax.jit
def kernel(x, W_router, W_gate, W_up, W_down):
    orig_shape = x.shape
    x2 = x.reshape(-1, DIM)
    n = x2.shape[0]

    idx, wts, eid, na = pl.pallas_call(
        _router_kernel,
        out_shape=(
            jax.ShapeDtypeStruct((2, n), jnp.int32),
            jax.ShapeDtypeStruct((2, n), jnp.float32),
            jax.ShapeDtypeStruct((1, NUM_EXPERTS), jnp.int32),
            jax.ShapeDtypeStruct((1, 1), jnp.int32),
        ),
    )(x2, W_router)

    grid_spec = pltpu.PrefetchScalarGridSpec(
        num_scalar_prefetch=2,
        grid=(NUM_EXPERTS,),
        in_specs=[
            pl.BlockSpec((n, DIM), lambda i, eid, na: (0, 0)),
            pl.BlockSpec((2, n), lambda i, eid, na: (0, 0)),
            pl.BlockSpec((2, n), lambda i, eid, na: (0, 0)),
            pl.BlockSpec((1, HIDDEN, DIM), lambda i, eid, na: (eid[0, i], 0, 0)),
            pl.BlockSpec((1, HIDDEN, DIM), lambda i, eid, na: (eid[0, i], 0, 0)),
            pl.BlockSpec((1, DIM, HIDDEN), lambda i, eid, na: (eid[0, i], 0, 0)),
        ],
        out_specs=pl.BlockSpec((n, DIM), lambda i, eid, na: (0, 0)),
    )
    out = pl.pallas_call(
        _moe_kernel,
        grid_spec=grid_spec,
        out_shape=jax.ShapeDtypeStruct((n, DIM), jnp.float32),
        compiler_params=pltpu.CompilerParams(
            dimension_semantics=("arbitrary",),
        ),
    )(eid, na, x2, idx, wts, W_gate, W_up, W_down)

    return out.reshape(orig_shape)
